# baseline (speedup 1.0000x reference)
import functools

import jax
import jax.numpy as jnp
from jax import lax
from jax.experimental import pallas as pl
from jax.experimental.pallas import tpu as pltpu
from jax.experimental.pallas import tpu_sc as plsc

B = 16384
F = 26
V = 1000012
D = 16
H1 = 32
H2 = 32
BF = B * F

NC = 2
NS = 16
NW = NC * NS
PER_W = BF // NW
CHUNK = 128
GPO = 8
STEP_ROWS = CHUNK * GPO
STEPS = PER_W // STEP_ROWS
IDX_ROWS = PER_W // CHUNK

VPAD = 1000064
V16 = (V + 15) // 16
SLAB = 1024
NSLAB_FULL = VPAD // SLAB
TAIL_COLS = VPAD - NSLAB_FULL * SLAB
SLABS_PER_W = (NSLAB_FULL + NW - 1) // NW

TC_BLK = 1024


def _sc_detile(w2t3):
  mesh = plsc.VectorSubcoreMesh(core_axis_name="c", subcore_axis_name="s")

  @functools.partial(
      pl.kernel,
      out_type=jax.ShapeDtypeStruct((VPAD * D // 128, 128), jnp.float32),
      mesh=mesh,
      compiler_params=pltpu.CompilerParams(use_tc_tiling_on_sc=True,
                                           needs_layout_passes=False),
      scratch_types=[
          pltpu.VMEM((16, SLAB), jnp.float32),
          pltpu.VMEM((SLAB * D // 128, 128), jnp.float32),
          pltpu.SemaphoreType.DMA,
      ],
  )
  def k(src_hbm, out_hbm, buf_v, st_v, sem):
    wid = lax.axis_index("s") * NC + lax.axis_index("c")
    lanes = lax.iota(jnp.int32, 16)

    def do_slab(cbase, width):
      cbase = pl.multiple_of(cbase, 128)
      obase = pl.multiple_of(cbase * D // 128, 8)
      for tr in range(2):
        pltpu.async_copy(src_hbm.at[tr, :, pl.ds(cbase, width)],
                         buf_v.at[pl.ds(8 * tr, 8), pl.ds(0, width)], sem)
      for tr in range(2):
        pltpu.make_async_copy(src_hbm.at[tr, :, pl.ds(cbase, width)],
                              buf_v.at[pl.ds(8 * tr, 8), pl.ds(0, width)],
                              sem).wait()
      def asm(t, _):
        colb = jnp.full((16,), t * 32, jnp.int32)
        for u in range(32):
          vals = plsc.load_gather(buf_v, [lanes, colb + u])
          st_v[t * 4 + (u >> 3), pl.ds((u & 7) * 16, 16)] = vals
        return 0

      lax.fori_loop(0, width // 32, asm, 0)
      pltpu.sync_copy(st_v.at[pl.ds(0, width * D // 128)],
                      out_hbm.at[pl.ds(obase, width * D // 128)])

    def step(t, _):
      s = wid * SLABS_PER_W + t

      @pl.when(s < NSLAB_FULL)
      def _():
        do_slab(s * SLAB, SLAB)

      return 0

    lax.fori_loop(0, SLABS_PER_W, step, 0)

    @pl.when(wid == 0)
    def _():
      do_slab(NSLAB_FULL * SLAB, TAIL_COLS)

  return k(w2t3)


def _sc_gather(xi2d, xihi2d, w2tab, w1tab):
  mesh = plsc.VectorSubcoreMesh(core_axis_name="c", subcore_axis_name="s")

  @functools.partial(
      pl.kernel,
      out_type=(
          jax.ShapeDtypeStruct((BF, D), jnp.float32),
          jax.ShapeDtypeStruct((BF,), jnp.float32),
      ),
      mesh=mesh,
      compiler_params=pltpu.CompilerParams(use_tc_tiling_on_sc=False,
                                           needs_layout_passes=False),
      scratch_types=[
          pltpu.VMEM((IDX_ROWS, CHUNK), jnp.int32),
          pltpu.VMEM((IDX_ROWS, CHUNK), jnp.int32),
          pltpu.VMEM((STEP_ROWS, D), jnp.float32),
          pltpu.VMEM((STEP_ROWS, D), jnp.float32),
          pltpu.VMEM((STEP_ROWS,), jnp.float32),
          pltpu.SemaphoreType.DMA,
          pltpu.SemaphoreType.DMA,
      ],
  )
  def k(xi_hbm, xihi_hbm, w2_hbm, w1t_hbm, emb_hbm, w1o_hbm,
        idx_v, idxhi_v, rows_v, w1blk_v, w1val_v, sem2, sem1):
    wid = lax.axis_index("s") * NC + lax.axis_index("c")
    pltpu.sync_copy(xi_hbm.at[pl.ds(wid * IDX_ROWS, IDX_ROWS)], idx_v)
    pltpu.sync_copy(xihi_hbm.at[pl.ds(wid * IDX_ROWS, IDX_ROWS)], idxhi_v)

    def step(o, _):
      descs = []
      for j in range(GPO):
        r = o * GPO + j
        descs.append(pltpu.async_copy(
            w2_hbm.at[idx_v.at[r]], rows_v.at[pl.ds(j * CHUNK, CHUNK)], sem2))
        descs.append(pltpu.async_copy(
            w1t_hbm.at[idxhi_v.at[r]], w1blk_v.at[pl.ds(j * CHUNK, CHUNK)],
            sem1))
      for d in descs:
        d.wait()
      for j in range(GPO):
        r = o * GPO + j
        for g in range(CHUNK // 16):
          col = idx_v[r, pl.ds(g * 16, 16)] & 15
          row = lax.iota(jnp.int32, 16) + (j * CHUNK + g * 16)
          w1val_v[pl.ds(j * CHUNK + g * 16, 16)] = plsc.load_gather(
              w1blk_v, [row, col])
      gbase = wid * PER_W + o * STEP_ROWS
      pltpu.sync_copy(rows_v, emb_hbm.at[pl.ds(gbase, STEP_ROWS)])
      pltpu.sync_copy(w1val_v, w1o_hbm.at[pl.ds(gbase, STEP_ROWS)])
      return 0

    lax.fori_loop(0, STEPS, step, 0)

  return k(xi2d, xihi2d, w2tab, w1tab)


def _tc_body(emb_ref, w1_ref, l1w_ref, l1b_ref, l2w_ref, l2b_ref, s_ref,
             bias_ref, out_ref):
  e = emb_ref[...]
  h1 = jnp.dot(e, l1w_ref[...], preferred_element_type=jnp.float32)
  h1 = jnp.maximum(h1 + l1b_ref[...], 0.0)
  h2 = jnp.dot(h1, l2w_ref[...], preferred_element_type=jnp.float32)
  h2 = jnp.maximum(h2 + l2b_ref[...], 0.0)
  fm_sum = jnp.dot(e, s_ref[...], preferred_element_type=jnp.float32)
  second = 0.5 * (jnp.sum(fm_sum * fm_sum, axis=1) - jnp.sum(e * e, axis=1))
  first = jnp.sum(w1_ref[...], axis=1)
  out_ref[...] = first + second + jnp.sum(h2, axis=1) + bias_ref[0]


def _tc_dense(emb, w1g, l1_w, l1_b, l2_w, l2_b, bias):
  smat = jnp.tile(jnp.eye(D, dtype=jnp.float32), (F, 1))
  grid = B // TC_BLK
  return pl.pallas_call(
      _tc_body,
      grid=(grid,),
      in_specs=[
          pl.BlockSpec((TC_BLK, F * D), lambda i: (i, 0)),
          pl.BlockSpec((TC_BLK, F), lambda i: (i, 0)),
          pl.BlockSpec((F * D, H1), lambda i: (0, 0)),
          pl.BlockSpec((1, H1), lambda i: (0, 0)),
          pl.BlockSpec((H1, H2), lambda i: (0, 0)),
          pl.BlockSpec((1, H2), lambda i: (0, 0)),
          pl.BlockSpec((F * D, D), lambda i: (0, 0)),
          pl.BlockSpec(memory_space=pltpu.SMEM),
      ],
      out_specs=pl.BlockSpec((TC_BLK,), lambda i: (i,)),
      out_shape=jax.ShapeDtypeStruct((B,), jnp.float32),
  )(emb, w1g, l1_w, l1_b.reshape(1, H1), l2_w, l2_b.reshape(1, H2), smat,
    bias)


def kernel(X, W1, W2, bias, l1_w, l1_b, l2_w, l2_b):
  xi = X.reshape(BF).astype(jnp.int32)
  xi2d = xi.reshape(BF // CHUNK, CHUNK)
  xihi2d = (xi >> 4).reshape(BF // CHUNK, CHUNK)
  w1tab = jnp.pad(W1.reshape(-1), (0, V16 * 16 - V)).reshape(V16, D)
  w2t3 = jnp.pad(W2.T, ((0, 0), (0, VPAD - V))).reshape(2, 8, VPAD)
  rowtab = _sc_detile(w2t3).reshape(VPAD, D)
  emb, w1g = _sc_gather(xi2d, xihi2d, rowtab, w1tab)
  return _tc_dense(emb.reshape(B, F * D), w1g.reshape(B, F),
                   l1_w, l1_b, l2_w, l2_b, bias)


# speedup vs baseline: 1.0015x; 1.0015x over previous
import functools

import jax
import jax.numpy as jnp
from jax import lax
from jax.experimental import pallas as pl
from jax.experimental.pallas import tpu as pltpu
from jax.experimental.pallas import tpu_sc as plsc

B = 16384
F = 26
V = 1000012
D = 16
H1 = 32
H2 = 32
BF = B * F

NC = 2
NS = 16
NW = NC * NS
PER_W = BF // NW
CHUNK = 128
GPO = 8
STEP_ROWS = CHUNK * GPO
STEPS = PER_W // STEP_ROWS
IDX_ROWS = PER_W // CHUNK

VPAD = 1000064
V16 = (V + 15) // 16
SLAB = 1024
NSLAB_FULL = VPAD // SLAB
TAIL_COLS = VPAD - NSLAB_FULL * SLAB
SLABS_PER_W = (NSLAB_FULL + NW - 1) // NW

TC_BLK = 1024


def _sc_detile(w2t3):
  mesh = plsc.VectorSubcoreMesh(core_axis_name="c", subcore_axis_name="s")

  @functools.partial(
      pl.kernel,
      out_type=jax.ShapeDtypeStruct((VPAD * D // 128, 128), jnp.float32),
      mesh=mesh,
      compiler_params=pltpu.CompilerParams(use_tc_tiling_on_sc=True,
                                           needs_layout_passes=False),
      scratch_types=[
          pltpu.VMEM((16, SLAB + 7), jnp.float32),
          pltpu.VMEM((SLAB * D // 128, 128), jnp.float32),
          pltpu.SemaphoreType.DMA,
      ],
  )
  def k(src_hbm, out_hbm, buf_v, st_v, sem):
    wid = lax.axis_index("s") * NC + lax.axis_index("c")
    lanes = lax.iota(jnp.int32, 16)

    def do_slab(cbase, width):
      cbase = pl.multiple_of(cbase, 128)
      obase = pl.multiple_of(cbase * D // 128, 8)
      for tr in range(2):
        pltpu.async_copy(src_hbm.at[tr, :, pl.ds(cbase, width)],
                         buf_v.at[pl.ds(8 * tr, 8), pl.ds(0, width)], sem)
      for tr in range(2):
        pltpu.make_async_copy(src_hbm.at[tr, :, pl.ds(cbase, width)],
                              buf_v.at[pl.ds(8 * tr, 8), pl.ds(0, width)],
                              sem).wait()
      def asm(t, _):
        colb = jnp.full((16,), t * 32, jnp.int32)
        for u in range(32):
          vals = plsc.load_gather(buf_v, [lanes, colb + u])
          st_v[t * 4 + (u >> 3), pl.ds((u & 7) * 16, 16)] = vals
        return 0

      lax.fori_loop(0, width // 32, asm, 0)
      pltpu.sync_copy(st_v.at[pl.ds(0, width * D // 128)],
                      out_hbm.at[pl.ds(obase, width * D // 128)])

    def step(t, _):
      s = wid * SLABS_PER_W + t

      @pl.when(s < NSLAB_FULL)
      def _():
        do_slab(s * SLAB, SLAB)

      return 0

    lax.fori_loop(0, SLABS_PER_W, step, 0)

    @pl.when(wid == 0)
    def _():
      do_slab(NSLAB_FULL * SLAB, TAIL_COLS)

  return k(w2t3)


def _sc_gather(xi2d, xihi2d, w2tab, w1tab):
  mesh = plsc.VectorSubcoreMesh(core_axis_name="c", subcore_axis_name="s")

  @functools.partial(
      pl.kernel,
      out_type=(
          jax.ShapeDtypeStruct((BF, D), jnp.float32),
          jax.ShapeDtypeStruct((BF,), jnp.float32),
      ),
      mesh=mesh,
      compiler_params=pltpu.CompilerParams(use_tc_tiling_on_sc=False,
                                           needs_layout_passes=False),
      scratch_types=[
          pltpu.VMEM((IDX_ROWS, CHUNK), jnp.int32),
          pltpu.VMEM((IDX_ROWS, CHUNK), jnp.int32),
          pltpu.VMEM((STEP_ROWS, D), jnp.float32),
          pltpu.VMEM((STEP_ROWS, D), jnp.float32),
          pltpu.VMEM((STEP_ROWS,), jnp.float32),
          pltpu.SemaphoreType.DMA,
          pltpu.SemaphoreType.DMA,
      ],
  )
  def k(xi_hbm, xihi_hbm, w2_hbm, w1t_hbm, emb_hbm, w1o_hbm,
        idx_v, idxhi_v, rows_v, w1blk_v, w1val_v, sem2, sem1):
    wid = lax.axis_index("s") * NC + lax.axis_index("c")
    pltpu.sync_copy(xi_hbm.at[pl.ds(wid * IDX_ROWS, IDX_ROWS)], idx_v)
    pltpu.sync_copy(xihi_hbm.at[pl.ds(wid * IDX_ROWS, IDX_ROWS)], idxhi_v)

    def step(o, _):
      descs = []
      for j in range(GPO):
        r = o * GPO + j
        descs.append(pltpu.async_copy(
            w2_hbm.at[idx_v.at[r]], rows_v.at[pl.ds(j * CHUNK, CHUNK)], sem2))
        descs.append(pltpu.async_copy(
            w1t_hbm.at[idxhi_v.at[r]], w1blk_v.at[pl.ds(j * CHUNK, CHUNK)],
            sem1))
      for d in descs:
        d.wait()
      for j in range(GPO):
        r = o * GPO + j
        for g in range(CHUNK // 16):
          col = idx_v[r, pl.ds(g * 16, 16)] & 15
          row = lax.iota(jnp.int32, 16) + (j * CHUNK + g * 16)
          w1val_v[pl.ds(j * CHUNK + g * 16, 16)] = plsc.load_gather(
              w1blk_v, [row, col])
      gbase = wid * PER_W + o * STEP_ROWS
      pltpu.sync_copy(rows_v, emb_hbm.at[pl.ds(gbase, STEP_ROWS)])
      pltpu.sync_copy(w1val_v, w1o_hbm.at[pl.ds(gbase, STEP_ROWS)])
      return 0

    lax.fori_loop(0, STEPS, step, 0)

  return k(xi2d, xihi2d, w2tab, w1tab)


def _tc_body(emb_ref, w1_ref, l1w_ref, l1b_ref, l2w_ref, l2b_ref, s_ref,
             bias_ref, out_ref):
  e = emb_ref[...]
  h1 = jnp.dot(e, l1w_ref[...], preferred_element_type=jnp.float32)
  h1 = jnp.maximum(h1 + l1b_ref[...], 0.0)
  h2 = jnp.dot(h1, l2w_ref[...], preferred_element_type=jnp.float32)
  h2 = jnp.maximum(h2 + l2b_ref[...], 0.0)
  fm_sum = jnp.dot(e, s_ref[...], preferred_element_type=jnp.float32)
  second = 0.5 * (jnp.sum(fm_sum * fm_sum, axis=1) - jnp.sum(e * e, axis=1))
  first = jnp.sum(w1_ref[...], axis=1)
  out_ref[...] = first + second + jnp.sum(h2, axis=1) + bias_ref[0]


def _tc_dense(emb, w1g, l1_w, l1_b, l2_w, l2_b, bias):
  smat = jnp.tile(jnp.eye(D, dtype=jnp.float32), (F, 1))
  grid = B // TC_BLK
  return pl.pallas_call(
      _tc_body,
      grid=(grid,),
      in_specs=[
          pl.BlockSpec((TC_BLK, F * D), lambda i: (i, 0)),
          pl.BlockSpec((TC_BLK, F), lambda i: (i, 0)),
          pl.BlockSpec((F * D, H1), lambda i: (0, 0)),
          pl.BlockSpec((1, H1), lambda i: (0, 0)),
          pl.BlockSpec((H1, H2), lambda i: (0, 0)),
          pl.BlockSpec((1, H2), lambda i: (0, 0)),
          pl.BlockSpec((F * D, D), lambda i: (0, 0)),
          pl.BlockSpec(memory_space=pltpu.SMEM),
      ],
      out_specs=pl.BlockSpec((TC_BLK,), lambda i: (i,)),
      out_shape=jax.ShapeDtypeStruct((B,), jnp.float32),
  )(emb, w1g, l1_w, l1_b.reshape(1, H1), l2_w, l2_b.reshape(1, H2), smat,
    bias)


def kernel(X, W1, W2, bias, l1_w, l1_b, l2_w, l2_b):
  xi = X.reshape(BF).astype(jnp.int32)
  xi2d = xi.reshape(BF // CHUNK, CHUNK)
  xihi2d = (xi >> 4).reshape(BF // CHUNK, CHUNK)
  w1tab = jnp.pad(W1.reshape(-1), (0, V16 * 16 - V)).reshape(V16, D)
  w2t3 = jnp.pad(W2.T, ((0, 0), (0, VPAD - V))).reshape(2, 8, VPAD)
  rowtab = _sc_detile(w2t3).reshape(VPAD, D)
  emb, w1g = _sc_gather(xi2d, xihi2d, rowtab, w1tab)
  return _tc_dense(emb.reshape(B, F * D), w1g.reshape(B, F),
                   l1_w, l1_b, l2_w, l2_b, bias)


# speedup vs baseline: 1.3335x; 1.3316x over previous
import functools

import jax
import jax.numpy as jnp
from jax import lax
from jax.experimental import pallas as pl
from jax.experimental.pallas import tpu as pltpu
from jax.experimental.pallas import tpu_sc as plsc

B = 16384
F = 26
V = 1000012
D = 16
H1 = 32
H2 = 32
BF = B * F

NC = 2
NS = 16
NW = NC * NS
PER_W = BF // NW
CHUNK = 128
GPO = 8
STEP_ROWS = CHUNK * GPO
STEPS = PER_W // STEP_ROWS
IDX_ROWS = PER_W // CHUNK

VPAD = 1000064
V16 = (V + 15) // 16
SLAB = 1024
NSLAB_FULL = VPAD // SLAB
TAIL_COLS = VPAD - NSLAB_FULL * SLAB
SLABS_PER_W = (NSLAB_FULL + NW - 1) // NW

TC_BLK = 1024


def _sc_detile(w2t3):
  mesh = plsc.VectorSubcoreMesh(core_axis_name="c", subcore_axis_name="s")

  @functools.partial(
      pl.kernel,
      out_type=jax.ShapeDtypeStruct((VPAD * D // 128, 128), jnp.float32),
      mesh=mesh,
      compiler_params=pltpu.CompilerParams(use_tc_tiling_on_sc=True,
                                           needs_layout_passes=False),
      scratch_types=[
          pltpu.VMEM((16, SLAB + 7), jnp.float32),
          pltpu.VMEM((SLAB * D // 128, 128), jnp.float32),
          pltpu.SemaphoreType.DMA,
      ],
  )
  def k(src_hbm, out_hbm, buf_v, st_v, sem):
    wid = lax.axis_index("s") * NC + lax.axis_index("c")
    lanes = lax.iota(jnp.int32, 16)

    def do_slab(cbase, width):
      cbase = pl.multiple_of(cbase, 128)
      obase = pl.multiple_of(cbase * D // 128, 8)
      for tr in range(2):
        pltpu.async_copy(src_hbm.at[tr, :, pl.ds(cbase, width)],
                         buf_v.at[pl.ds(8 * tr, 8), pl.ds(0, width)], sem)
      for tr in range(2):
        pltpu.make_async_copy(src_hbm.at[tr, :, pl.ds(cbase, width)],
                              buf_v.at[pl.ds(8 * tr, 8), pl.ds(0, width)],
                              sem).wait()
      def asm(t, _):
        colb = jnp.full((16,), t * 32, jnp.int32)
        vals = [plsc.load_gather(buf_v, [lanes, colb + u]) for u in range(32)]
        for u in range(32):
          st_v[t * 4 + (u >> 3), pl.ds((u & 7) * 16, 16)] = vals[u]
        return 0

      lax.fori_loop(0, width // 32, asm, 0)
      pltpu.sync_copy(st_v.at[pl.ds(0, width * D // 128)],
                      out_hbm.at[pl.ds(obase, width * D // 128)])

    def step(t, _):
      s = wid * SLABS_PER_W + t

      @pl.when(s < NSLAB_FULL)
      def _():
        do_slab(s * SLAB, SLAB)

      return 0

    lax.fori_loop(0, SLABS_PER_W, step, 0)

    @pl.when(wid == 0)
    def _():
      do_slab(NSLAB_FULL * SLAB, TAIL_COLS)

  return k(w2t3)


def _sc_gather(xi2d, xihi2d, w2tab, w1tab):
  mesh = plsc.VectorSubcoreMesh(core_axis_name="c", subcore_axis_name="s")

  @functools.partial(
      pl.kernel,
      out_type=(
          jax.ShapeDtypeStruct((BF, D), jnp.float32),
          jax.ShapeDtypeStruct((BF,), jnp.float32),
      ),
      mesh=mesh,
      compiler_params=pltpu.CompilerParams(use_tc_tiling_on_sc=False,
                                           needs_layout_passes=False),
      scratch_types=[
          pltpu.VMEM((IDX_ROWS, CHUNK), jnp.int32),
          pltpu.VMEM((IDX_ROWS, CHUNK), jnp.int32),
          pltpu.VMEM((STEP_ROWS, D), jnp.float32),
          pltpu.VMEM((STEP_ROWS, D), jnp.float32),
          pltpu.VMEM((STEP_ROWS,), jnp.float32),
          pltpu.SemaphoreType.DMA,
          pltpu.SemaphoreType.DMA,
      ],
  )
  def k(xi_hbm, xihi_hbm, w2_hbm, w1t_hbm, emb_hbm, w1o_hbm,
        idx_v, idxhi_v, rows_v, w1blk_v, w1val_v, sem2, sem1):
    wid = lax.axis_index("s") * NC + lax.axis_index("c")
    pltpu.sync_copy(xi_hbm.at[pl.ds(wid * IDX_ROWS, IDX_ROWS)], idx_v)
    pltpu.sync_copy(xihi_hbm.at[pl.ds(wid * IDX_ROWS, IDX_ROWS)], idxhi_v)

    def step(o, _):
      descs = []
      for j in range(GPO):
        r = o * GPO + j
        descs.append(pltpu.async_copy(
            w2_hbm.at[idx_v.at[r]], rows_v.at[pl.ds(j * CHUNK, CHUNK)], sem2))
        descs.append(pltpu.async_copy(
            w1t_hbm.at[idxhi_v.at[r]], w1blk_v.at[pl.ds(j * CHUNK, CHUNK)],
            sem1))
      for d in descs:
        d.wait()
      for j in range(GPO):
        r = o * GPO + j
        for g in range(CHUNK // 16):
          col = idx_v[r, pl.ds(g * 16, 16)] & 15
          row = lax.iota(jnp.int32, 16) + (j * CHUNK + g * 16)
          w1val_v[pl.ds(j * CHUNK + g * 16, 16)] = plsc.load_gather(
              w1blk_v, [row, col])
      gbase = wid * PER_W + o * STEP_ROWS
      pltpu.sync_copy(rows_v, emb_hbm.at[pl.ds(gbase, STEP_ROWS)])
      pltpu.sync_copy(w1val_v, w1o_hbm.at[pl.ds(gbase, STEP_ROWS)])
      return 0

    lax.fori_loop(0, STEPS, step, 0)

  return k(xi2d, xihi2d, w2tab, w1tab)


def _tc_body(emb_ref, w1_ref, l1w_ref, l1b_ref, l2w_ref, l2b_ref, s_ref,
             bias_ref, out_ref):
  e = emb_ref[...]
  h1 = jnp.dot(e, l1w_ref[...], preferred_element_type=jnp.float32)
  h1 = jnp.maximum(h1 + l1b_ref[...], 0.0)
  h2 = jnp.dot(h1, l2w_ref[...], preferred_element_type=jnp.float32)
  h2 = jnp.maximum(h2 + l2b_ref[...], 0.0)
  fm_sum = jnp.dot(e, s_ref[...], preferred_element_type=jnp.float32)
  second = 0.5 * (jnp.sum(fm_sum * fm_sum, axis=1) - jnp.sum(e * e, axis=1))
  first = jnp.sum(w1_ref[...], axis=1)
  out_ref[...] = first + second + jnp.sum(h2, axis=1) + bias_ref[0]


def _tc_dense(emb, w1g, l1_w, l1_b, l2_w, l2_b, bias):
  smat = jnp.tile(jnp.eye(D, dtype=jnp.float32), (F, 1))
  grid = B // TC_BLK
  return pl.pallas_call(
      _tc_body,
      grid=(grid,),
      in_specs=[
          pl.BlockSpec((TC_BLK, F * D), lambda i: (i, 0)),
          pl.BlockSpec((TC_BLK, F), lambda i: (i, 0)),
          pl.BlockSpec((F * D, H1), lambda i: (0, 0)),
          pl.BlockSpec((1, H1), lambda i: (0, 0)),
          pl.BlockSpec((H1, H2), lambda i: (0, 0)),
          pl.BlockSpec((1, H2), lambda i: (0, 0)),
          pl.BlockSpec((F * D, D), lambda i: (0, 0)),
          pl.BlockSpec(memory_space=pltpu.SMEM),
      ],
      out_specs=pl.BlockSpec((TC_BLK,), lambda i: (i,)),
      out_shape=jax.ShapeDtypeStruct((B,), jnp.float32),
  )(emb, w1g, l1_w, l1_b.reshape(1, H1), l2_w, l2_b.reshape(1, H2), smat,
    bias)


def kernel(X, W1, W2, bias, l1_w, l1_b, l2_w, l2_b):
  xi = X.reshape(BF).astype(jnp.int32)
  xi2d = xi.reshape(BF // CHUNK, CHUNK)
  xihi2d = (xi >> 4).reshape(BF // CHUNK, CHUNK)
  w1tab = jnp.pad(W1.reshape(-1), (0, V16 * 16 - V)).reshape(V16, D)
  w2t3 = jnp.pad(W2.T, ((0, 0), (0, VPAD - V))).reshape(2, 8, VPAD)
  rowtab = _sc_detile(w2t3).reshape(VPAD, D)
  emb, w1g = _sc_gather(xi2d, xihi2d, rowtab, w1tab)
  return _tc_dense(emb.reshape(B, F * D), w1g.reshape(B, F),
                   l1_w, l1_b, l2_w, l2_b, bias)


# speedup vs baseline: 1.5258x; 1.1442x over previous
import functools

import jax
import jax.numpy as jnp
from jax import lax
from jax.experimental import pallas as pl
from jax.experimental.pallas import tpu as pltpu
from jax.experimental.pallas import tpu_sc as plsc

B = 16384
F = 26
V = 1000012
D = 16
H1 = 32
H2 = 32
BF = B * F

NC = 2
NS = 16
NW = NC * NS
PER_W = BF // NW
CHUNK = 128
GPO = 8
STEP_ROWS = CHUNK * GPO
STEPS = PER_W // STEP_ROWS
IDX_ROWS = PER_W // CHUNK

VPAD = 1000064
V16 = (V + 15) // 16
SLAB = 1024
NSLAB_FULL = VPAD // SLAB
TAIL_COLS = VPAD - NSLAB_FULL * SLAB
SLABS_PER_W = (NSLAB_FULL + NW - 1) // NW

TC_BLK = 1024


def _sc_detile(w2t3):
  mesh = plsc.VectorSubcoreMesh(core_axis_name="c", subcore_axis_name="s")

  @functools.partial(
      pl.kernel,
      out_type=jax.ShapeDtypeStruct((VPAD * D // 128, 128), jnp.float32),
      mesh=mesh,
      compiler_params=pltpu.CompilerParams(use_tc_tiling_on_sc=True,
                                           needs_layout_passes=False),
      scratch_types=[
          pltpu.VMEM((2, 16, SLAB), jnp.float32),
          pltpu.VMEM((2, SLAB * D // 128, 128), jnp.float32),
          pltpu.SemaphoreType.DMA,
          pltpu.SemaphoreType.DMA,
          pltpu.SemaphoreType.DMA,
          pltpu.SemaphoreType.DMA,
      ],
  )
  def k(src_hbm, out_hbm, buf_v, st_v, semi_a, semi_b, semo_a, semo_b):
    wid = lax.axis_index("s") * NC + lax.axis_index("c")
    lanes = lax.iota(jnp.int32, 16)
    semi = (semi_a, semi_b)
    semo = (semo_a, semo_b)

    def in_refs(t, p):
      cb = pl.multiple_of((wid * SLABS_PER_W + t) * SLAB, 128)
      return [(src_hbm.at[tr, :, pl.ds(cb, SLAB)],
               buf_v.at[p, pl.ds(8 * tr, 8), :]) for tr in range(2)]

    def out_refs(t, p):
      ob = pl.multiple_of((wid * SLABS_PER_W + t) * (SLAB * D // 128), 8)
      return (st_v.at[p], out_hbm.at[pl.ds(ob, SLAB * D // 128)])

    def valid(t):
      return jnp.logical_and(
          jnp.logical_and(t >= 0, t < SLABS_PER_W),
          wid * SLABS_PER_W + t < NSLAB_FULL)

    def fire_in(t, p):
      @pl.when(valid(t))
      def _():
        for s, d in in_refs(t, p):
          pltpu.async_copy(s, d, semi[p])

    def drain_in(t, p):
      @pl.when(valid(t))
      def _():
        for s, d in in_refs(t, p):
          pltpu.make_async_copy(s, d, semi[p]).wait()

    def fire_out(t, p):
      @pl.when(valid(t))
      def _():
        s, d = out_refs(t, p)
        pltpu.async_copy(s, d, semo[p])

    def drain_out(t, p):
      @pl.when(valid(t))
      def _():
        s, d = out_refs(t, p)
        pltpu.make_async_copy(s, d, semo[p]).wait()

    def asm_body(p, g, width):
      colb = jnp.full((16,), g * 32, jnp.int32)
      vals = [plsc.load_gather(buf_v.at[p], [lanes, colb + u])
              for u in range(32)]
      for u in range(32):
        st_v[p, g * 4 + (u >> 3), pl.ds((u & 7) * 16, 16)] = vals[u]

    def assemble(t, p):
      @pl.when(valid(t))
      def _():
        def asm(g, _):
          asm_body(p, g, SLAB)
          return 0

        lax.fori_loop(0, SLAB // 32, asm, 0)

    fire_in(0, 0)

    def step2(kk, _):
      e = 2 * kk
      fire_in(e + 1, 1)
      drain_in(e, 0)
      drain_out(e - 2, 0)
      assemble(e, 0)
      fire_out(e, 0)
      fire_in(e + 2, 0)
      drain_in(e + 1, 1)
      drain_out(e - 1, 1)
      assemble(e + 1, 1)
      fire_out(e + 1, 1)
      return 0

    lax.fori_loop(0, (SLABS_PER_W + 1) // 2, step2, 0)
    drain_out(SLABS_PER_W - 1, (SLABS_PER_W - 1) % 2)

    @pl.when(wid == 0)
    def _():
      cb = NSLAB_FULL * SLAB
      for tr in range(2):
        pltpu.async_copy(src_hbm.at[tr, :, pl.ds(cb, TAIL_COLS)],
                         buf_v.at[0, pl.ds(8 * tr, 8), pl.ds(0, TAIL_COLS)],
                         semi_a)
      for tr in range(2):
        pltpu.make_async_copy(src_hbm.at[tr, :, pl.ds(cb, TAIL_COLS)],
                              buf_v.at[0, pl.ds(8 * tr, 8),
                                       pl.ds(0, TAIL_COLS)], semi_a).wait()

      def asm(g, _):
        asm_body(0, g, TAIL_COLS)
        return 0

      lax.fori_loop(0, TAIL_COLS // 32, asm, 0)
      pltpu.sync_copy(st_v.at[0, pl.ds(0, TAIL_COLS * D // 128)],
                      out_hbm.at[pl.ds(cb * D // 128, TAIL_COLS * D // 128)])

  return k(w2t3)


def _sc_gather(xi2d, xihi2d, w2tab, w1tab):
  mesh = plsc.VectorSubcoreMesh(core_axis_name="c", subcore_axis_name="s")

  @functools.partial(
      pl.kernel,
      out_type=(
          jax.ShapeDtypeStruct((BF, D), jnp.float32),
          jax.ShapeDtypeStruct((BF,), jnp.float32),
      ),
      mesh=mesh,
      compiler_params=pltpu.CompilerParams(use_tc_tiling_on_sc=False,
                                           needs_layout_passes=False),
      scratch_types=[
          pltpu.VMEM((IDX_ROWS, CHUNK), jnp.int32),
          pltpu.VMEM((IDX_ROWS, CHUNK), jnp.int32),
          pltpu.VMEM((STEP_ROWS, D), jnp.float32),
          pltpu.VMEM((STEP_ROWS, D), jnp.float32),
          pltpu.VMEM((STEP_ROWS,), jnp.float32),
          pltpu.SemaphoreType.DMA,
          pltpu.SemaphoreType.DMA,
      ],
  )
  def k(xi_hbm, xihi_hbm, w2_hbm, w1t_hbm, emb_hbm, w1o_hbm,
        idx_v, idxhi_v, rows_v, w1blk_v, w1val_v, sem2, sem1):
    wid = lax.axis_index("s") * NC + lax.axis_index("c")
    pltpu.sync_copy(xi_hbm.at[pl.ds(wid * IDX_ROWS, IDX_ROWS)], idx_v)
    pltpu.sync_copy(xihi_hbm.at[pl.ds(wid * IDX_ROWS, IDX_ROWS)], idxhi_v)

    def step(o, _):
      descs = []
      for j in range(GPO):
        r = o * GPO + j
        descs.append(pltpu.async_copy(
            w2_hbm.at[idx_v.at[r]], rows_v.at[pl.ds(j * CHUNK, CHUNK)], sem2))
        descs.append(pltpu.async_copy(
            w1t_hbm.at[idxhi_v.at[r]], w1blk_v.at[pl.ds(j * CHUNK, CHUNK)],
            sem1))
      for d in descs:
        d.wait()
      for j in range(GPO):
        r = o * GPO + j
        for g in range(CHUNK // 16):
          col = idx_v[r, pl.ds(g * 16, 16)] & 15
          row = lax.iota(jnp.int32, 16) + (j * CHUNK + g * 16)
          w1val_v[pl.ds(j * CHUNK + g * 16, 16)] = plsc.load_gather(
              w1blk_v, [row, col])
      gbase = wid * PER_W + o * STEP_ROWS
      pltpu.sync_copy(rows_v, emb_hbm.at[pl.ds(gbase, STEP_ROWS)])
      pltpu.sync_copy(w1val_v, w1o_hbm.at[pl.ds(gbase, STEP_ROWS)])
      return 0

    lax.fori_loop(0, STEPS, step, 0)

  return k(xi2d, xihi2d, w2tab, w1tab)


def _tc_body(emb_ref, w1_ref, l1w_ref, l1b_ref, l2w_ref, l2b_ref, s_ref,
             bias_ref, out_ref):
  e = emb_ref[...]
  h1 = jnp.dot(e, l1w_ref[...], preferred_element_type=jnp.float32)
  h1 = jnp.maximum(h1 + l1b_ref[...], 0.0)
  h2 = jnp.dot(h1, l2w_ref[...], preferred_element_type=jnp.float32)
  h2 = jnp.maximum(h2 + l2b_ref[...], 0.0)
  fm_sum = jnp.dot(e, s_ref[...], preferred_element_type=jnp.float32)
  second = 0.5 * (jnp.sum(fm_sum * fm_sum, axis=1) - jnp.sum(e * e, axis=1))
  first = jnp.sum(w1_ref[...], axis=1)
  out_ref[...] = first + second + jnp.sum(h2, axis=1) + bias_ref[0]


def _tc_dense(emb, w1g, l1_w, l1_b, l2_w, l2_b, bias):
  smat = jnp.tile(jnp.eye(D, dtype=jnp.float32), (F, 1))
  grid = B // TC_BLK
  return pl.pallas_call(
      _tc_body,
      grid=(grid,),
      in_specs=[
          pl.BlockSpec((TC_BLK, F * D), lambda i: (i, 0)),
          pl.BlockSpec((TC_BLK, F), lambda i: (i, 0)),
          pl.BlockSpec((F * D, H1), lambda i: (0, 0)),
          pl.BlockSpec((1, H1), lambda i: (0, 0)),
          pl.BlockSpec((H1, H2), lambda i: (0, 0)),
          pl.BlockSpec((1, H2), lambda i: (0, 0)),
          pl.BlockSpec((F * D, D), lambda i: (0, 0)),
          pl.BlockSpec(memory_space=pltpu.SMEM),
      ],
      out_specs=pl.BlockSpec((TC_BLK,), lambda i: (i,)),
      out_shape=jax.ShapeDtypeStruct((B,), jnp.float32),
  )(emb, w1g, l1_w, l1_b.reshape(1, H1), l2_w, l2_b.reshape(1, H2), smat,
    bias)


def kernel(X, W1, W2, bias, l1_w, l1_b, l2_w, l2_b):
  xi = X.reshape(BF).astype(jnp.int32)
  xi2d = xi.reshape(BF // CHUNK, CHUNK)
  xihi2d = (xi >> 4).reshape(BF // CHUNK, CHUNK)
  w1tab = jnp.pad(W1.T, ((0, 0), (0, VPAD - V))).reshape(VPAD // D, D)
  w2t3 = jnp.pad(W2.T, ((0, 0), (0, VPAD - V))).reshape(2, 8, VPAD)
  rowtab = _sc_detile(w2t3).reshape(VPAD, D)
  emb, w1g = _sc_gather(xi2d, xihi2d, rowtab, w1tab)
  return _tc_dense(emb.reshape(B, F * D), w1g.reshape(B, F),
                   l1_w, l1_b, l2_w, l2_b, bias)


# speedup vs baseline: 1.5288x; 1.0020x over previous
import functools

import jax
import jax.numpy as jnp
from jax import lax
from jax.experimental import pallas as pl
from jax.experimental.pallas import tpu as pltpu
from jax.experimental.pallas import tpu_sc as plsc

B = 16384
F = 26
V = 1000012
D = 16
H1 = 32
H2 = 32
BF = B * F

NC = 2
NS = 16
NW = NC * NS
PER_W = BF // NW
CHUNK = 128
GPO = 8
STEP_ROWS = CHUNK * GPO
STEPS = PER_W // STEP_ROWS
IDX_ROWS = PER_W // CHUNK

VPAD = 1000064
V16 = (V + 15) // 16
SLAB = 1024
NSLAB_FULL = VPAD // SLAB
TAIL_COLS = VPAD - NSLAB_FULL * SLAB
SLABS_PER_W = (NSLAB_FULL + NW - 1) // NW

TC_BLK = 1024


def _sc_detile(w2t3):
  mesh = plsc.VectorSubcoreMesh(core_axis_name="c", subcore_axis_name="s")

  @functools.partial(
      pl.kernel,
      out_type=jax.ShapeDtypeStruct((VPAD * D // 128, 128), jnp.float32),
      mesh=mesh,
      compiler_params=pltpu.CompilerParams(use_tc_tiling_on_sc=True,
                                           needs_layout_passes=False),
      scratch_types=[
          pltpu.VMEM((2, 16, SLAB), jnp.float32),
          pltpu.VMEM((2, SLAB * D // 128, 128), jnp.float32),
          pltpu.SemaphoreType.DMA,
          pltpu.SemaphoreType.DMA,
          pltpu.SemaphoreType.DMA,
          pltpu.SemaphoreType.DMA,
      ],
  )
  def k(src_hbm, out_hbm, buf_v, st_v, semi_a, semi_b, semo_a, semo_b):
    wid = lax.axis_index("s") * NC + lax.axis_index("c")
    lanes = lax.iota(jnp.int32, 16)
    semi = (semi_a, semi_b)
    semo = (semo_a, semo_b)

    def in_refs(t, p):
      cb = pl.multiple_of((wid * SLABS_PER_W + t) * SLAB, 128)
      return [(src_hbm.at[tr, :, pl.ds(cb, SLAB)],
               buf_v.at[p, pl.ds(8 * tr, 8), :]) for tr in range(2)]

    def out_refs(t, p):
      ob = pl.multiple_of((wid * SLABS_PER_W + t) * (SLAB * D // 128), 8)
      return (st_v.at[p], out_hbm.at[pl.ds(ob, SLAB * D // 128)])

    def valid(t):
      return jnp.logical_and(
          jnp.logical_and(t >= 0, t < SLABS_PER_W),
          wid * SLABS_PER_W + t < NSLAB_FULL)

    def fire_in(t, p):
      @pl.when(valid(t))
      def _():
        for s, d in in_refs(t, p):
          pltpu.async_copy(s, d, semi[p])

    def drain_in(t, p):
      @pl.when(valid(t))
      def _():
        for s, d in in_refs(t, p):
          pltpu.make_async_copy(s, d, semi[p]).wait()

    def fire_out(t, p):
      @pl.when(valid(t))
      def _():
        s, d = out_refs(t, p)
        pltpu.async_copy(s, d, semo[p])

    def drain_out(t, p):
      @pl.when(valid(t))
      def _():
        s, d = out_refs(t, p)
        pltpu.make_async_copy(s, d, semo[p]).wait()

    def asm_body(p, g, width):
      colb = jnp.full((16,), g * 32, jnp.int32)
      vals = [plsc.load_gather(buf_v.at[p], [lanes, colb + u])
              for u in range(32)]
      for u in range(32):
        st_v[p, g * 4 + (u >> 3), pl.ds((u & 7) * 16, 16)] = vals[u]

    def assemble(t, p):
      @pl.when(valid(t))
      def _():
        def asm(g, _):
          asm_body(p, g, SLAB)
          return 0

        lax.fori_loop(0, SLAB // 32, asm, 0)

    fire_in(0, 0)

    def step2(kk, _):
      e = 2 * kk
      fire_in(e + 1, 1)
      drain_in(e, 0)
      drain_out(e - 2, 0)
      assemble(e, 0)
      fire_out(e, 0)
      fire_in(e + 2, 0)
      drain_in(e + 1, 1)
      drain_out(e - 1, 1)
      assemble(e + 1, 1)
      fire_out(e + 1, 1)
      return 0

    lax.fori_loop(0, (SLABS_PER_W + 1) // 2, step2, 0)
    drain_out(SLABS_PER_W - 1, (SLABS_PER_W - 1) % 2)

    @pl.when(wid == 0)
    def _():
      cb = NSLAB_FULL * SLAB
      for tr in range(2):
        pltpu.async_copy(src_hbm.at[tr, :, pl.ds(cb, TAIL_COLS)],
                         buf_v.at[0, pl.ds(8 * tr, 8), pl.ds(0, TAIL_COLS)],
                         semi_a)
      for tr in range(2):
        pltpu.make_async_copy(src_hbm.at[tr, :, pl.ds(cb, TAIL_COLS)],
                              buf_v.at[0, pl.ds(8 * tr, 8),
                                       pl.ds(0, TAIL_COLS)], semi_a).wait()

      def asm(g, _):
        asm_body(0, g, TAIL_COLS)
        return 0

      lax.fori_loop(0, TAIL_COLS // 32, asm, 0)
      pltpu.sync_copy(st_v.at[0, pl.ds(0, TAIL_COLS * D // 128)],
                      out_hbm.at[pl.ds(cb * D // 128, TAIL_COLS * D // 128)])

  return k(w2t3)


def _sc_gather(xi2d, xihi2d, w2tab, w1tab):
  mesh = plsc.VectorSubcoreMesh(core_axis_name="c", subcore_axis_name="s")

  @functools.partial(
      pl.kernel,
      out_type=(
          jax.ShapeDtypeStruct((BF, D), jnp.float32),
          jax.ShapeDtypeStruct((BF,), jnp.float32),
      ),
      mesh=mesh,
      compiler_params=pltpu.CompilerParams(use_tc_tiling_on_sc=False,
                                           needs_layout_passes=False),
      scratch_types=[
          pltpu.VMEM((IDX_ROWS, CHUNK), jnp.int32),
          pltpu.VMEM((IDX_ROWS, CHUNK), jnp.int32),
          pltpu.VMEM((STEP_ROWS, D), jnp.float32),
          pltpu.VMEM((STEP_ROWS, D), jnp.float32),
          pltpu.VMEM((STEP_ROWS,), jnp.float32),
          pltpu.SemaphoreType.DMA,
          pltpu.SemaphoreType.DMA,
      ],
  )
  def k(xi_hbm, xihi_hbm, w2_hbm, w1t_hbm, emb_hbm, w1o_hbm,
        idx_v, idxhi_v, rows_v, w1blk_v, w1val_v, sem2, sem1):
    wid = lax.axis_index("s") * NC + lax.axis_index("c")
    pltpu.sync_copy(xi_hbm.at[pl.ds(wid * IDX_ROWS, IDX_ROWS)], idx_v)
    pltpu.sync_copy(xihi_hbm.at[pl.ds(wid * IDX_ROWS, IDX_ROWS)], idxhi_v)

    def step(o, _):
      descs = []
      for j in range(GPO):
        r = o * GPO + j
        descs.append(pltpu.async_copy(
            w2_hbm.at[idx_v.at[r]], rows_v.at[pl.ds(j * CHUNK, CHUNK)], sem2))
        descs.append(pltpu.async_copy(
            w1t_hbm.at[idxhi_v.at[r]], w1blk_v.at[pl.ds(j * CHUNK, CHUNK)],
            sem1))
      for d in descs:
        d.wait()
      for j in range(GPO):
        r = o * GPO + j
        for g in range(CHUNK // 16):
          col = idx_v[r, pl.ds(g * 16, 16)] & 15
          row = lax.iota(jnp.int32, 16) + (j * CHUNK + g * 16)
          w1val_v[pl.ds(j * CHUNK + g * 16, 16)] = plsc.load_gather(
              w1blk_v, [row, col])
      gbase = wid * PER_W + o * STEP_ROWS
      pltpu.sync_copy(rows_v, emb_hbm.at[pl.ds(gbase, STEP_ROWS)])
      pltpu.sync_copy(w1val_v, w1o_hbm.at[pl.ds(gbase, STEP_ROWS)])
      return 0

    lax.fori_loop(0, STEPS, step, 0)

  return k(xi2d, xihi2d, w2tab, w1tab)


def _tc_body(emb_ref, w1_ref, l1w_ref, l1b_ref, l2w_ref, l2b_ref, s_ref,
             bias_ref, out_ref):
  e = emb_ref[...]
  h1 = jnp.dot(e, l1w_ref[...], preferred_element_type=jnp.float32)
  h1 = jnp.maximum(h1 + l1b_ref[...], 0.0)
  h2 = jnp.dot(h1, l2w_ref[...], preferred_element_type=jnp.float32)
  h2 = jnp.maximum(h2 + l2b_ref[...], 0.0)
  fm_sum = jnp.dot(e, s_ref[...], preferred_element_type=jnp.float32)
  second = 0.5 * (jnp.sum(fm_sum * fm_sum, axis=1) - jnp.sum(e * e, axis=1))
  first = jnp.sum(w1_ref[...], axis=1)
  out_ref[...] = first + second + jnp.sum(h2, axis=1) + bias_ref[0]


def _tc_dense(emb, w1g, l1_w, l1_b, l2_w, l2_b, bias):
  smat = jnp.tile(jnp.eye(D, dtype=jnp.float32), (F, 1))
  grid = B // TC_BLK
  return pl.pallas_call(
      _tc_body,
      grid=(grid,),
      in_specs=[
          pl.BlockSpec((TC_BLK, F * D), lambda i: (i, 0)),
          pl.BlockSpec((TC_BLK, F), lambda i: (i, 0)),
          pl.BlockSpec((F * D, H1), lambda i: (0, 0)),
          pl.BlockSpec((1, H1), lambda i: (0, 0)),
          pl.BlockSpec((H1, H2), lambda i: (0, 0)),
          pl.BlockSpec((1, H2), lambda i: (0, 0)),
          pl.BlockSpec((F * D, D), lambda i: (0, 0)),
          pl.BlockSpec(memory_space=pltpu.SMEM),
      ],
      out_specs=pl.BlockSpec((TC_BLK,), lambda i: (i,)),
      out_shape=jax.ShapeDtypeStruct((B,), jnp.float32),
  )(emb, w1g, l1_w, l1_b.reshape(1, H1), l2_w, l2_b.reshape(1, H2), smat,
    bias)


def kernel(X, W1, W2, bias, l1_w, l1_b, l2_w, l2_b):
  w2t3 = jnp.pad(W2.T, ((0, 0), (0, VPAD - V))).reshape(2, 8, VPAD)
  rowtab = _sc_detile(w2t3).reshape(VPAD, D)
  xi = X.reshape(BF).astype(jnp.int32)
  xi2d = xi.reshape(BF // CHUNK, CHUNK)
  xihi2d = (xi >> 4).reshape(BF // CHUNK, CHUNK)
  w1p = jax.lax.optimization_barrier(jnp.pad(W1.T, ((0, 0), (0, VPAD - V))))
  w1tab = w1p.reshape(VPAD // D, D)
  emb, w1g = _sc_gather(xi2d, xihi2d, rowtab, w1tab)
  return _tc_dense(emb.reshape(B, F * D), w1g.reshape(B, F),
                   l1_w, l1_b, l2_w, l2_b, bias)
